# baseline (speedup 1.0000x reference)
import jax
import jax.numpy as jnp
from jax import lax
from jax.experimental import pallas as pl
from jax.experimental.pallas import tpu as pltpu
from jax.experimental.pallas import tpu_sc as plsc

D = 64
DP = 128
SCALE = 8.0

NC = 2
NS = 16
NW = NC * NS

NB = 16384
NT = 50
G = 128
C = 256
B_PER_W = NB // NW
NCH = NT * (B_PER_W // C)


def _body(xt_hbm, ep_hbm, out_hbm,
          idx_v, g0_v, g1_v, s0_v, s1_v,
          gsem0, gsem1, wsem0, wsem1):
    wid = lax.axis_index("s") * NC + lax.axis_index("c")
    b0 = wid * B_PER_W

    gbuf = (g0_v, g1_v)
    sbuf = (s0_v, s1_v)
    gsem = (gsem0, gsem1)
    wsem = (wsem0, wsem1)

    pltpu.sync_copy(xt_hbm.at[:, pl.ds(b0, B_PER_W)], idx_v)

    def fire(c, bf):
        t = c // 2
        off = (c % 2) * C
        for k in range(C // G):
            pltpu.async_copy(
                ep_hbm.at[idx_v.at[t, pl.ds(off + k * G, G)]],
                gbuf[bf].at[pl.ds(k * G, G)],
                gsem[bf],
            )

    def wait_gather(bf):
        pltpu.make_async_copy(
            ep_hbm.at[pl.ds(0, C)], gbuf[bf], gsem[bf]
        ).wait()

    def transpose_scale(bf):
        g = gbuf[bf]
        s = sbuf[bf]

        @plsc.parallel_loop(0, C * D // 16, step=1, unroll=2)
        def _(v):
            jhi = v >> 7
            bsub = (v >> 6) & 1
            jlo = (v >> 3) & 7
            blo0 = (v & 7) * 16
            lane = lax.iota(jnp.int32, 16)
            rows = bsub * G + blo0 + lane
            cols = jnp.zeros((16,), jnp.int32) + (jhi * 8 + jlo)
            vals = plsc.load_gather(g, [rows, cols])
            s[jhi, bsub, jlo, pl.ds(blo0, 16)] = vals * SCALE

    def start_wb(c, bf):
        t = c // 2
        bhi0 = wid * 4 + (c % 2) * 2
        for jhi in range(8):
            pltpu.async_copy(
                sbuf[bf].at[jhi],
                out_hbm.at[t, jhi, pl.ds(bhi0, 2)],
                wsem[bf],
            )

    def wait_wb(bf):
        pltpu.make_async_copy(
            sbuf[bf], out_hbm.at[0, :, pl.ds(0, 2)], wsem[bf]
        ).wait()

    fire(0, 0)
    fire(1, 1)
    for bf in range(2):
        wait_gather(bf)
        transpose_scale(bf)
        fire(2 + bf, bf)
        start_wb(bf, bf)

    def step(o, _):
        for bf in range(2):
            c = 2 * o + bf
            wait_gather(bf)
            wait_wb(bf)
            transpose_scale(bf)
            fire(c + 2, bf)
            start_wb(c, bf)
        return 0

    lax.fori_loop(1, NCH // 2 - 1, step, 0)

    for bf in range(2):
        c = NCH - 2 + bf
        wait_gather(bf)
        wait_wb(bf)
        transpose_scale(bf)
        start_wb(c, bf)
    for bf in range(2):
        wait_wb(bf)


def kernel(x, E):
    xt = x.T.astype(jnp.int32)
    ep = jnp.pad(E, ((0, 0), (0, DP - D)))
    mesh = plsc.VectorSubcoreMesh(
        core_axis_name="c", subcore_axis_name="s", num_cores=NC, num_subcores=NS
    )
    out5 = pl.kernel(
        _body,
        out_type=jax.ShapeDtypeStruct((NT, 8, NB // G, 8, G), jnp.float32),
        mesh=mesh,
        scratch_types=[
            pltpu.VMEM((NT, B_PER_W), jnp.int32),
            pltpu.VMEM((C, DP), jnp.float32),
            pltpu.VMEM((C, DP), jnp.float32),
            pltpu.VMEM((8, 2, 8, G), jnp.float32),
            pltpu.VMEM((8, 2, 8, G), jnp.float32),
            pltpu.SemaphoreType.DMA,
            pltpu.SemaphoreType.DMA,
            pltpu.SemaphoreType.DMA,
            pltpu.SemaphoreType.DMA,
        ],
        compiler_params=pltpu.CompilerParams(
            use_tc_tiling_on_sc=False, needs_layout_passes=False
        ),
    )(xt, ep)
    return out5.transpose(2, 4, 0, 1, 3).reshape(NB, NT, D)


# speedup vs baseline: 1.1513x; 1.1513x over previous
import jax
import jax.numpy as jnp
from jax import lax
from jax.experimental import pallas as pl
from jax.experimental.pallas import tpu as pltpu
from jax.experimental.pallas import tpu_sc as plsc

D = 64
DP = 128
SCALE = 8.0

NC = 2
NS = 16
NW = NC * NS

NB = 16384
NT = 50
G = 128
C = 256
B_PER_W = NB // NW
NCH = NT * (B_PER_W // C)

S_T = 8 * 128 * 8 * 128
S_JHI = 128 * 8 * 128
S_BHI = 8 * 128
OUT_FLAT = NT * S_T


def _body(xt_hbm, ep_hbm, out_hbm,
          idx_v, g0_v, g1_v, s0_v, s1_v,
          gsem0, gsem1, wsem0, wsem1):
    wid = lax.axis_index("s") * NC + lax.axis_index("c")
    b0 = wid * B_PER_W

    gbuf = (g0_v, g1_v)
    sbuf = (s0_v, s1_v)
    gsem = (gsem0, gsem1)
    wsem = (wsem0, wsem1)

    lane = lax.iota(jnp.int32, 16)
    kbase = ((lane >> 3) << 11) + ((lane & 7) << 7)

    pltpu.sync_copy(xt_hbm.at[:, pl.ds(b0, B_PER_W)], idx_v)

    def fire(c, bf):
        t = c // 2
        off = (c % 2) * C
        for k in range(C // G):
            pltpu.async_copy(
                ep_hbm.at[idx_v.at[t, pl.ds(off + k * G, G)]],
                gbuf[bf].at[pl.ds(k * G, G)],
                gsem[bf],
            )

    def wait_gather(bf):
        pltpu.make_async_copy(
            ep_hbm.at[pl.ds(0, C)], gbuf[bf], gsem[bf]
        ).wait()

    def transpose_scale(bf):
        g = gbuf[bf]
        s = sbuf[bf]

        @plsc.parallel_loop(0, C * D // 16, step=1, unroll=4)
        def _(v):
            b = v >> 2
            q = v & 3
            soff = (q << 12) + ((b >> 7) << 10) + (b & 127)
            vals = g[b, pl.ds(q * 16, 16)]
            plsc.store_scatter(s, [kbase + soff], vals * SCALE)

    def start_wb(c, bf):
        t = c // 2
        base = t * S_T + (wid * 4 + (c % 2) * 2) * S_BHI
        for jhi in range(8):
            pltpu.async_copy(
                sbuf[bf].at[pl.ds(jhi * 2048, 2048)],
                out_hbm.at[pl.ds(base + jhi * S_JHI, 2048)],
                wsem[bf],
            )

    def wait_wb(bf):
        pltpu.make_async_copy(
            sbuf[bf], out_hbm.at[pl.ds(0, C * D)], wsem[bf]
        ).wait()

    fire(0, 0)
    fire(1, 1)
    for bf in range(2):
        wait_gather(bf)
        transpose_scale(bf)
        fire(2 + bf, bf)
        start_wb(bf, bf)

    def step(o, _):
        for bf in range(2):
            c = 2 * o + bf
            wait_gather(bf)
            wait_wb(bf)
            transpose_scale(bf)
            fire(c + 2, bf)
            start_wb(c, bf)
        return 0

    lax.fori_loop(1, NCH // 2 - 1, step, 0)

    for bf in range(2):
        c = NCH - 2 + bf
        wait_gather(bf)
        wait_wb(bf)
        transpose_scale(bf)
        start_wb(c, bf)
    for bf in range(2):
        wait_wb(bf)


def kernel(x, E):
    xt = x.T.astype(jnp.int32)
    ep = jnp.pad(E, ((0, 0), (0, DP - D)))
    mesh = plsc.VectorSubcoreMesh(
        core_axis_name="c", subcore_axis_name="s", num_cores=NC, num_subcores=NS
    )
    out1 = pl.kernel(
        _body,
        out_type=jax.ShapeDtypeStruct((OUT_FLAT,), jnp.float32),
        mesh=mesh,
        scratch_types=[
            pltpu.VMEM((NT, B_PER_W), jnp.int32),
            pltpu.VMEM((C, DP), jnp.float32),
            pltpu.VMEM((C, DP), jnp.float32),
            pltpu.VMEM((C * D,), jnp.float32),
            pltpu.VMEM((C * D,), jnp.float32),
            pltpu.SemaphoreType.DMA,
            pltpu.SemaphoreType.DMA,
            pltpu.SemaphoreType.DMA,
            pltpu.SemaphoreType.DMA,
        ],
        compiler_params=pltpu.CompilerParams(
            use_tc_tiling_on_sc=False, needs_layout_passes=False
        ),
    )(xt, ep)
    out5 = out1.reshape(NT, 8, NB // G, 8, G)
    return out5.transpose(2, 4, 0, 1, 3).reshape(NB, NT, D)
